# baseline (speedup 1.0000x reference)
import jax
import jax.numpy as jnp
from jax.experimental import pallas as pl

BN = 128


def _vq_softmax_kernel(x_ref, cb_ref, csqr_ref, out_ref):
    x = x_ref[...]
    c = cb_ref[...]
    logits = jax.lax.dot_general(
        x, c, (((1,), (1,)), ((), ())), preferred_element_type=jnp.float32
    )
    logits = 2.0 * logits - csqr_ref[...]
    m = jnp.max(logits, axis=1, keepdims=True)
    e = jnp.exp(logits - m)
    s = jnp.sum(e, axis=1, keepdims=True)
    out_ref[...] = e * (1.0 / s)


def kernel(z_e_x, codebook):
    n_total = z_e_x.shape[0] * z_e_x.shape[1]
    d = z_e_x.shape[2]
    k = codebook.shape[0]
    x = z_e_x.reshape(n_total, d)
    csqr = jnp.sum(codebook * codebook, axis=1)[None, :]

    grid = (n_total // BN,)
    out = pl.pallas_call(
        _vq_softmax_kernel,
        grid=grid,
        in_specs=[
            pl.BlockSpec((BN, d), lambda i: (i, 0)),
            pl.BlockSpec((k, d), lambda i: (0, 0)),
            pl.BlockSpec((1, k), lambda i: (0, 0)),
        ],
        out_specs=pl.BlockSpec((BN, k), lambda i: (i, 0)),
        out_shape=jax.ShapeDtypeStruct((n_total, k), jnp.float32),
    )(x, codebook, csqr)
    return out


# speedup vs baseline: 1.4293x; 1.4293x over previous
import jax
import jax.numpy as jnp
from jax.experimental import pallas as pl

BN = 512


def _vq_softmax_kernel(x_ref, cb_ref, csqr_ref, out_ref):
    x = x_ref[...]
    c = cb_ref[...]
    logits = jax.lax.dot_general(
        x, c, (((1,), (1,)), ((), ())), preferred_element_type=jnp.float32
    )
    logits = 2.0 * logits - csqr_ref[...]
    m = jnp.max(logits, axis=1, keepdims=True)
    e = jnp.exp(logits - m)
    s = jnp.sum(e, axis=1, keepdims=True)
    out_ref[...] = e * (1.0 / s)


def kernel(z_e_x, codebook):
    n_total = z_e_x.shape[0] * z_e_x.shape[1]
    d = z_e_x.shape[2]
    k = codebook.shape[0]
    x = z_e_x.reshape(n_total, d)
    csqr = jnp.sum(codebook * codebook, axis=1)[None, :]

    grid = (n_total // BN,)
    out = pl.pallas_call(
        _vq_softmax_kernel,
        grid=grid,
        in_specs=[
            pl.BlockSpec((BN, d), lambda i: (i, 0)),
            pl.BlockSpec((k, d), lambda i: (0, 0)),
            pl.BlockSpec((1, k), lambda i: (0, 0)),
        ],
        out_specs=pl.BlockSpec((BN, k), lambda i: (i, 0)),
        out_shape=jax.ShapeDtypeStruct((n_total, k), jnp.float32),
    )(x, codebook, csqr)
    return out
